# baseline (speedup 1.0000x reference)
import functools

import jax
import jax.numpy as jnp
from jax import lax
from jax.experimental import pallas as pl
from jax.experimental.pallas import tpu as pltpu
from jax.experimental.pallas import tpu_sc as plsc

B = 16384
T = 50
D = 64
NC = 2
NS = 16
NW = NC * NS
RPW = B // NW
CHUNK = 16
NCHUNK = RPW // CHUNK


def _sc_gather_pool(news_flat, cat_idx, ent_idx, word_table, cat_table,
                    ent_table):
  mesh = plsc.VectorSubcoreMesh(core_axis_name="c", subcore_axis_name="s")

  buf = lambda shape, dtype: [pltpu.VMEM(shape, dtype)] * 2

  @functools.partial(
      pl.kernel,
      mesh=mesh,
      out_type=(
          jax.ShapeDtypeStruct((B, D), jnp.float32),
          jax.ShapeDtypeStruct((B, D), jnp.float32),
          jax.ShapeDtypeStruct((B, D), jnp.float32),
      ),
      compiler_params=pltpu.CompilerParams(use_tc_tiling_on_sc=False),
      scratch_types=[
          buf((CHUNK * T,), jnp.int32),
          buf((CHUNK * T, D), jnp.float32),
          buf((CHUNK, D), jnp.float32),
          [pltpu.SemaphoreType.DMA] * 2,
          [pltpu.SemaphoreType.DMA] * 2,
          pltpu.VMEM((RPW,), jnp.int32),
          pltpu.VMEM((RPW,), jnp.int32),
          pltpu.SemaphoreType.DMA,
          pltpu.SemaphoreType.DMA,
      ],
  )
  def body(news_r, wtab_r, cat_r, ent_r, ctab_r, etab_r,
           wvec_r, cvec_r, evec_r,
           idx_v, rows_v, acc_v, isem, gsem,
           cidx_v, eidx_v, cisem, cgsem):
    wid = lax.axis_index("s") * NC + lax.axis_index("c")
    base = wid * RPW

    pltpu.async_copy(cat_r.at[pl.ds(base, RPW)], cidx_v, cisem)
    pltpu.async_copy(ent_r.at[pl.ds(base, RPW)], eidx_v, cisem)

    def start_idx(j, p):
      off = (base + j * CHUNK) * T
      pltpu.async_copy(news_r.at[pl.ds(off, CHUNK * T)], idx_v[p], isem[p])

    def wait_idx(p):
      pltpu.make_async_copy(news_r.at[pl.ds(0, CHUNK * T)],
                            idx_v[p], isem[p]).wait()

    def fire_gather(p):
      pltpu.async_copy(wtab_r.at[idx_v[p]], rows_v[p], gsem[p])

    def wait_gather(p):
      pltpu.make_async_copy(wtab_r.at[idx_v[p]], rows_v[p], gsem[p]).wait()

    def compute_out(j, p):
      rows = rows_v[p]
      acc = acc_v[p]

      def row_body(r, rcarry):
        def t_body(t, accs):
          a0, a1, a2, a3 = accs
          src = r * T + t
          a0 = a0 + rows[src, 0:16]
          a1 = a1 + rows[src, 16:32]
          a2 = a2 + rows[src, 32:48]
          a3 = a3 + rows[src, 48:64]
          return (a0, a1, a2, a3)

        z = jnp.zeros((16,), jnp.float32)
        a0, a1, a2, a3 = lax.fori_loop(0, T, t_body, (z, z, z, z), unroll=5)
        acc[r, 0:16] = a0
        acc[r, 16:32] = a1
        acc[r, 32:48] = a2
        acc[r, 48:64] = a3
        return rcarry

      lax.fori_loop(0, CHUNK, row_body, 0)
      pltpu.sync_copy(acc, wvec_r.at[pl.ds(base + j * CHUNK, CHUNK)])

    start_idx(0, 0)
    wait_idx(0)
    fire_gather(0)
    start_idx(1, 1)


    def pair_body(jj, carry):
      j0 = 2 * jj
      j1 = j0 + 1
      wait_idx(1)
      fire_gather(1)
      wait_gather(0)

      @pl.when(j0 + 2 < NCHUNK)
      def _():
        start_idx(j0 + 2, 0)

      compute_out(j0, 0)

      @pl.when(j0 + 2 < NCHUNK)
      def _():
        wait_idx(0)
        fire_gather(0)

      wait_gather(1)

      @pl.when(j1 + 2 < NCHUNK)
      def _():
        start_idx(j1 + 2, 1)

      compute_out(j1, 1)
      return carry

    lax.fori_loop(0, NCHUNK // 2, pair_body, 0)

    crows = rows_v[0].at[pl.ds(0, RPW)]
    erows = rows_v[1].at[pl.ds(0, RPW)]
    pltpu.make_async_copy(cat_r.at[pl.ds(0, RPW)], cidx_v, cisem).wait()
    pltpu.make_async_copy(ent_r.at[pl.ds(0, RPW)], eidx_v, cisem).wait()
    pltpu.async_copy(ctab_r.at[cidx_v], crows, cgsem)
    pltpu.async_copy(etab_r.at[eidx_v], erows, cgsem)
    pltpu.make_async_copy(ctab_r.at[cidx_v], crows, cgsem).wait()
    pltpu.make_async_copy(etab_r.at[eidx_v], erows, cgsem).wait()
    pltpu.sync_copy(crows, cvec_r.at[pl.ds(base, RPW)])
    pltpu.sync_copy(erows, evec_r.at[pl.ds(base, RPW)])

  return body(news_flat, word_table, cat_idx, ent_idx, cat_table, ent_table)


TC_BLK = 2048


def _tc_fuse(wsum, news, cvec, evec, W, b):
  def body(ws_r, news_r, cv_r, ev_r, w_r, b_r, out_r):
    mask = (news_r[...] != 0).astype(jnp.float32)
    cnt = jnp.sum(mask, axis=1, keepdims=True)
    wv = ws_r[...] / (cnt + 1e-08)
    dot = functools.partial(
        lax.dot_general,
        dimension_numbers=(((1,), (0,)), ((), ())),
        precision=lax.Precision.HIGHEST,
        preferred_element_type=jnp.float32,
    )
    acc = dot(wv, w_r[0:D, :])
    acc = acc + dot(cv_r[...], w_r[D:2 * D, :])
    acc = acc + dot(ev_r[...], w_r[2 * D:3 * D, :])
    out_r[...] = jnp.maximum(acc + b_r[...], 0.0)

  return pl.pallas_call(
      body,
      grid=(B // TC_BLK,),
      in_specs=[
          pl.BlockSpec((TC_BLK, D), lambda i: (i, 0)),
          pl.BlockSpec((TC_BLK, T), lambda i: (i, 0)),
          pl.BlockSpec((TC_BLK, D), lambda i: (i, 0)),
          pl.BlockSpec((TC_BLK, D), lambda i: (i, 0)),
          pl.BlockSpec((3 * D, D), lambda i: (0, 0)),
          pl.BlockSpec((1, D), lambda i: (0, 0)),
      ],
      out_specs=pl.BlockSpec((TC_BLK, D), lambda i: (i, 0)),
      out_shape=jax.ShapeDtypeStruct((B, D), jnp.float32),
  )(wsum, news, cvec, evec, W, b.reshape(1, D))


def kernel(news_input, cat_input, ent_input, word_table, cat_table, ent_table, W, b):
  news_input = news_input.astype(jnp.int32)
  news_flat = news_input.reshape(-1)
  cat_input = cat_input.astype(jnp.int32)
  ent_input = ent_input.astype(jnp.int32)
  wsum, cvec, evec = _sc_gather_pool(
      news_flat, cat_input, ent_input, word_table, cat_table, ent_table)
  return _tc_fuse(wsum, news_input, cvec, evec, W, b)


# speedup vs baseline: 1.0929x; 1.0929x over previous
import functools

import jax
import jax.numpy as jnp
from jax import lax
from jax.experimental import pallas as pl
from jax.experimental.pallas import tpu as pltpu
from jax.experimental.pallas import tpu_sc as plsc

B = 16384
T = 50
D = 64
NC = 2
NS = 16
NW = NC * NS
RPW = B // NW
CHUNK = 16
NCHUNK = RPW // CHUNK


def _sc_word_pool(news_flat, word_table):
  mesh = plsc.VectorSubcoreMesh(core_axis_name="c", subcore_axis_name="s")

  buf = lambda shape, dtype: [pltpu.VMEM(shape, dtype)] * 2

  @functools.partial(
      pl.kernel,
      mesh=mesh,
      out_type=jax.ShapeDtypeStruct((B, D), jnp.float32),
      compiler_params=pltpu.CompilerParams(use_tc_tiling_on_sc=False),
      scratch_types=[
          buf((CHUNK * T,), jnp.int32),
          buf((CHUNK * T, D), jnp.float32),
          buf((CHUNK, D), jnp.float32),
          [pltpu.SemaphoreType.DMA] * 2,
          [pltpu.SemaphoreType.DMA] * 2,
      ],
  )
  def body(news_r, wtab_r, wvec_r, idx_v, rows_v, acc_v, isem, gsem):
    wid = lax.axis_index("s") * NC + lax.axis_index("c")
    base = wid * RPW

    def start_idx(j, p):
      off = (base + j * CHUNK) * T
      pltpu.async_copy(news_r.at[pl.ds(off, CHUNK * T)], idx_v[p], isem[p])

    def wait_idx(p):
      pltpu.make_async_copy(news_r.at[pl.ds(0, CHUNK * T)],
                            idx_v[p], isem[p]).wait()

    def fire_gather(p):
      pltpu.async_copy(wtab_r.at[idx_v[p]], rows_v[p], gsem[p])

    def wait_gather(p):
      pltpu.make_async_copy(wtab_r.at[idx_v[p]], rows_v[p], gsem[p]).wait()

    def compute_out(j, p):
      rows = rows_v[p]
      acc = acc_v[p]

      def row_body(r, rcarry):
        def t_body(t, accs):
          a0, a1, a2, a3 = accs
          src = r * T + t
          a0 = a0 + rows[src, 0:16]
          a1 = a1 + rows[src, 16:32]
          a2 = a2 + rows[src, 32:48]
          a3 = a3 + rows[src, 48:64]
          return (a0, a1, a2, a3)

        z = jnp.zeros((16,), jnp.float32)
        a0, a1, a2, a3 = lax.fori_loop(0, T, t_body, (z, z, z, z), unroll=5)
        acc[r, 0:16] = a0
        acc[r, 16:32] = a1
        acc[r, 32:48] = a2
        acc[r, 48:64] = a3
        return rcarry

      lax.fori_loop(0, CHUNK, row_body, 0)
      pltpu.sync_copy(acc, wvec_r.at[pl.ds(base + j * CHUNK, CHUNK)])

    start_idx(0, 0)
    wait_idx(0)
    fire_gather(0)
    start_idx(1, 1)

    def pair_body(jj, carry):
      j0 = 2 * jj
      j1 = j0 + 1
      wait_idx(1)
      fire_gather(1)
      wait_gather(0)

      @pl.when(j0 + 2 < NCHUNK)
      def _():
        start_idx(j0 + 2, 0)

      compute_out(j0, 0)

      @pl.when(j0 + 2 < NCHUNK)
      def _():
        wait_idx(0)
        fire_gather(0)

      wait_gather(1)

      @pl.when(j1 + 2 < NCHUNK)
      def _():
        start_idx(j1 + 2, 1)

      compute_out(j1, 1)
      return carry

    lax.fori_loop(0, NCHUNK // 2, pair_body, 0)

  return body(news_flat, word_table)


def _sc_catent(dep, cat_idx, ent_idx, cat_table, ent_table):
  mesh = plsc.VectorSubcoreMesh(core_axis_name="c", subcore_axis_name="s")

  @functools.partial(
      pl.kernel,
      mesh=mesh,
      out_type=(
          jax.ShapeDtypeStruct((B, D), jnp.float32),
          jax.ShapeDtypeStruct((B, D), jnp.float32),
      ),
      compiler_params=pltpu.CompilerParams(use_tc_tiling_on_sc=False),
      scratch_types=[
          pltpu.VMEM((RPW,), jnp.int32),
          pltpu.VMEM((RPW,), jnp.int32),
          pltpu.VMEM((RPW, D), jnp.float32),
          pltpu.VMEM((RPW, D), jnp.float32),
          pltpu.SemaphoreType.DMA,
          pltpu.SemaphoreType.DMA,
      ],
  )
  def body(dep_r, cat_r, ent_r, ctab_r, etab_r, cvec_r, evec_r,
           cidx_v, eidx_v, crows_v, erows_v, isem, gsem):
    del dep_r
    wid = lax.axis_index("s") * NC + lax.axis_index("c")
    base = wid * RPW
    pltpu.async_copy(cat_r.at[pl.ds(base, RPW)], cidx_v, isem)
    pltpu.async_copy(ent_r.at[pl.ds(base, RPW)], eidx_v, isem)
    pltpu.make_async_copy(cat_r.at[pl.ds(0, RPW)], cidx_v, isem).wait()
    pltpu.make_async_copy(ent_r.at[pl.ds(0, RPW)], eidx_v, isem).wait()
    pltpu.async_copy(ctab_r.at[cidx_v], crows_v, gsem)
    pltpu.async_copy(etab_r.at[eidx_v], erows_v, gsem)
    pltpu.make_async_copy(ctab_r.at[cidx_v], crows_v, gsem).wait()
    pltpu.make_async_copy(etab_r.at[eidx_v], erows_v, gsem).wait()
    pltpu.sync_copy(crows_v, cvec_r.at[pl.ds(base, RPW)])
    pltpu.sync_copy(erows_v, evec_r.at[pl.ds(base, RPW)])

  return body(dep, cat_idx, ent_idx, cat_table, ent_table)


TC_BLK = 2048


def _tc_fuse(wsum, news, cvec, evec, W, b):
  def body(ws_r, news_r, cv_r, ev_r, w_r, b_r, out_r):
    mask = (news_r[...] != 0).astype(jnp.float32)
    cnt = jnp.sum(mask, axis=1, keepdims=True)
    wv = ws_r[...] / (cnt + 1e-08)
    dot = functools.partial(
        lax.dot_general,
        dimension_numbers=(((1,), (0,)), ((), ())),
        precision=lax.Precision.HIGHEST,
        preferred_element_type=jnp.float32,
    )
    acc = dot(wv, w_r[0:D, :])
    acc = acc + dot(cv_r[...], w_r[D:2 * D, :])
    acc = acc + dot(ev_r[...], w_r[2 * D:3 * D, :])
    out_r[...] = jnp.maximum(acc + b_r[...], 0.0)

  return pl.pallas_call(
      body,
      grid=(B // TC_BLK,),
      in_specs=[
          pl.BlockSpec((TC_BLK, D), lambda i: (i, 0)),
          pl.BlockSpec((TC_BLK, T), lambda i: (i, 0)),
          pl.BlockSpec((TC_BLK, D), lambda i: (i, 0)),
          pl.BlockSpec((TC_BLK, D), lambda i: (i, 0)),
          pl.BlockSpec((3 * D, D), lambda i: (0, 0)),
          pl.BlockSpec((1, D), lambda i: (0, 0)),
      ],
      out_specs=pl.BlockSpec((TC_BLK, D), lambda i: (i, 0)),
      out_shape=jax.ShapeDtypeStruct((B, D), jnp.float32),
  )(wsum, news, cvec, evec, W, b.reshape(1, D))


def kernel(news_input, cat_input, ent_input, word_table, cat_table, ent_table, W, b):
  news_input = news_input.astype(jnp.int32)
  news_flat = news_input.reshape(-1)
  cat_input = cat_input.astype(jnp.int32)
  ent_input = ent_input.astype(jnp.int32)
  wsum = _sc_word_pool(news_flat, word_table)
  cvec, evec = _sc_catent(wsum, cat_input, ent_input, cat_table, ent_table)
  return _tc_fuse(wsum, news_input, cvec, evec, W, b)


# speedup vs baseline: 1.1416x; 1.0445x over previous
import functools

import jax
import jax.numpy as jnp
from jax import lax
from jax.experimental import pallas as pl
from jax.experimental.pallas import tpu as pltpu
from jax.experimental.pallas import tpu_sc as plsc

B = 16384
T = 50
D = 64
NC = 2
NS = 16
NW = NC * NS
RPW = B // NW
CHUNK = 16
NCHUNK = RPW // CHUNK


def _sc_word_pool(news_flat, word_table):
  mesh = plsc.VectorSubcoreMesh(core_axis_name="c", subcore_axis_name="s")

  buf = lambda shape, dtype: [pltpu.VMEM(shape, dtype)] * 2

  @functools.partial(
      pl.kernel,
      mesh=mesh,
      out_type=jax.ShapeDtypeStruct((B, D), jnp.float32),
      compiler_params=pltpu.CompilerParams(use_tc_tiling_on_sc=False),
      scratch_types=[
          buf((CHUNK * T,), jnp.int32),
          buf((CHUNK * T, D), jnp.float32),
          buf((CHUNK, D), jnp.float32),
          [pltpu.SemaphoreType.DMA] * 2,
          [pltpu.SemaphoreType.DMA] * 2,
      ],
  )
  def body(news_r, wtab_r, wvec_r, idx_v, rows_v, acc_v, isem, gsem):
    wid = lax.axis_index("s") * NC + lax.axis_index("c")
    base = wid * RPW

    def start_idx(j, p):
      off = (base + j * CHUNK) * T
      pltpu.async_copy(news_r.at[pl.ds(off, CHUNK * T)], idx_v[p], isem[p])

    def wait_idx(p):
      pltpu.make_async_copy(news_r.at[pl.ds(0, CHUNK * T)],
                            idx_v[p], isem[p]).wait()

    def fire_gather(p):
      pltpu.async_copy(wtab_r.at[idx_v[p]], rows_v[p], gsem[p])

    def wait_gather(p):
      pltpu.make_async_copy(wtab_r.at[idx_v[p]], rows_v[p], gsem[p]).wait()

    def compute_out(j, p):
      rows = rows_v[p]
      acc = acc_v[p]

      def row_body(r, rcarry):
        def t_body(t, accs):
          a0, a1, a2, a3 = accs
          src = r * T + t
          a0 = a0 + rows[src, 0:16]
          a1 = a1 + rows[src, 16:32]
          a2 = a2 + rows[src, 32:48]
          a3 = a3 + rows[src, 48:64]
          return (a0, a1, a2, a3)

        z = jnp.zeros((16,), jnp.float32)
        a0, a1, a2, a3 = lax.fori_loop(0, T, t_body, (z, z, z, z), unroll=5)
        acc[r, 0:16] = a0
        acc[r, 16:32] = a1
        acc[r, 32:48] = a2
        acc[r, 48:64] = a3
        return rcarry

      lax.fori_loop(0, CHUNK, row_body, 0)
      pltpu.sync_copy(acc, wvec_r.at[pl.ds(base + j * CHUNK, CHUNK)])

    start_idx(0, 0)
    wait_idx(0)
    fire_gather(0)
    start_idx(1, 1)

    def pair_body(jj, carry):
      j0 = 2 * jj
      j1 = j0 + 1
      wait_idx(1)
      fire_gather(1)
      wait_gather(0)

      @pl.when(j0 + 2 < NCHUNK)
      def _():
        start_idx(j0 + 2, 0)

      compute_out(j0, 0)

      @pl.when(j0 + 2 < NCHUNK)
      def _():
        wait_idx(0)
        fire_gather(0)

      wait_gather(1)

      @pl.when(j1 + 2 < NCHUNK)
      def _():
        start_idx(j1 + 2, 1)

      compute_out(j1, 1)
      return carry

    lax.fori_loop(0, NCHUNK // 2, pair_body, 0)

  return body(news_flat, word_table)


def _sc_catent(dep, cat_idx, ent_idx, cat_table, ent_table):
  mesh = plsc.VectorSubcoreMesh(core_axis_name="c", subcore_axis_name="s")

  @functools.partial(
      pl.kernel,
      mesh=mesh,
      out_type=(
          jax.ShapeDtypeStruct((B, D), jnp.float32),
          jax.ShapeDtypeStruct((B, D), jnp.float32),
      ),
      compiler_params=pltpu.CompilerParams(use_tc_tiling_on_sc=False),
      scratch_types=[
          pltpu.VMEM((RPW,), jnp.int32),
          pltpu.VMEM((RPW,), jnp.int32),
          pltpu.VMEM((RPW, D), jnp.float32),
          pltpu.VMEM((RPW, D), jnp.float32),
          pltpu.SemaphoreType.DMA,
          pltpu.SemaphoreType.DMA,
      ],
  )
  def body(dep_r, cat_r, ent_r, ctab_r, etab_r, cvec_r, evec_r,
           cidx_v, eidx_v, crows_v, erows_v, isem, gsem):
    del dep_r
    wid = lax.axis_index("s") * NC + lax.axis_index("c")
    base = wid * RPW
    pltpu.async_copy(cat_r.at[pl.ds(base, RPW)], cidx_v, isem)
    pltpu.async_copy(ent_r.at[pl.ds(base, RPW)], eidx_v, isem)
    pltpu.make_async_copy(cat_r.at[pl.ds(0, RPW)], cidx_v, isem).wait()
    pltpu.make_async_copy(ent_r.at[pl.ds(0, RPW)], eidx_v, isem).wait()
    pltpu.async_copy(ctab_r.at[cidx_v], crows_v, gsem)
    pltpu.async_copy(etab_r.at[eidx_v], erows_v, gsem)
    pltpu.make_async_copy(ctab_r.at[cidx_v], crows_v, gsem).wait()
    pltpu.make_async_copy(etab_r.at[eidx_v], erows_v, gsem).wait()
    pltpu.sync_copy(crows_v, cvec_r.at[pl.ds(base, RPW)])
    pltpu.sync_copy(erows_v, evec_r.at[pl.ds(base, RPW)])

  return body(dep, cat_idx, ent_idx, cat_table, ent_table)


TC_BLK = 2048


def _tc_fuse(wsum, news, cvec, evec, W, b):
  def body(ws_r, news_r, cv_r, ev_r, w_r, b_r, out_r):
    mask = (news_r[...] != 0).astype(jnp.float32)
    cnt = jnp.sum(mask, axis=1, keepdims=True)
    wv = ws_r[...] / (cnt + 1e-08)
    dot = functools.partial(
        lax.dot_general,
        dimension_numbers=(((1,), (0,)), ((), ())),
        precision=lax.Precision.DEFAULT,
        preferred_element_type=jnp.float32,
    )
    acc = dot(wv, w_r[0:D, :])
    acc = acc + dot(cv_r[...], w_r[D:2 * D, :])
    acc = acc + dot(ev_r[...], w_r[2 * D:3 * D, :])
    out_r[...] = jnp.maximum(acc + b_r[...], 0.0)

  return pl.pallas_call(
      body,
      grid=(B // TC_BLK,),
      in_specs=[
          pl.BlockSpec((TC_BLK, D), lambda i: (i, 0)),
          pl.BlockSpec((TC_BLK, T), lambda i: (i, 0)),
          pl.BlockSpec((TC_BLK, D), lambda i: (i, 0)),
          pl.BlockSpec((TC_BLK, D), lambda i: (i, 0)),
          pl.BlockSpec((3 * D, D), lambda i: (0, 0)),
          pl.BlockSpec((1, D), lambda i: (0, 0)),
      ],
      out_specs=pl.BlockSpec((TC_BLK, D), lambda i: (i, 0)),
      out_shape=jax.ShapeDtypeStruct((B, D), jnp.float32),
  )(wsum, news, cvec, evec, W, b.reshape(1, D))


def kernel(news_input, cat_input, ent_input, word_table, cat_table, ent_table, W, b):
  news_input = news_input.astype(jnp.int32)
  news_flat = news_input.reshape(-1)
  cat_input = cat_input.astype(jnp.int32)
  ent_input = ent_input.astype(jnp.int32)
  wsum = _sc_word_pool(news_flat, word_table)
  cvec, evec = _sc_catent(wsum, cat_input, ent_input, cat_table, ent_table)
  return _tc_fuse(wsum, news_input, cvec, evec, W, b)


# speedup vs baseline: 1.1484x; 1.0060x over previous
import functools

import jax
import jax.numpy as jnp
from jax import lax
from jax.experimental import pallas as pl
from jax.experimental.pallas import tpu as pltpu
from jax.experimental.pallas import tpu_sc as plsc

B = 16384
T = 50
D = 64
NC = 2
NS = 16
NW = NC * NS
RPW = B // NW
CHUNK = 16
NCHUNK = RPW // CHUNK


def _sc_word_pool(news_flat, word_table):
  mesh = plsc.VectorSubcoreMesh(core_axis_name="c", subcore_axis_name="s")

  buf = lambda shape, dtype: [pltpu.VMEM(shape, dtype)] * 2

  @functools.partial(
      pl.kernel,
      mesh=mesh,
      out_type=jax.ShapeDtypeStruct((B, D), jnp.float32),
      compiler_params=pltpu.CompilerParams(use_tc_tiling_on_sc=False),
      scratch_types=[
          buf((CHUNK * T,), jnp.int32),
          buf((CHUNK * T, D), jnp.float32),
          buf((CHUNK, D), jnp.float32),
          [pltpu.SemaphoreType.DMA] * 2,
          [pltpu.SemaphoreType.DMA] * 2,
      ],
  )
  def body(news_r, wtab_r, wvec_r, idx_v, rows_v, acc_v, isem, gsem):
    wid = lax.axis_index("s") * NC + lax.axis_index("c")
    base = wid * RPW

    def start_idx(j, p):
      off = (base + j * CHUNK) * T
      pltpu.async_copy(news_r.at[pl.ds(off, CHUNK * T)], idx_v[p], isem[p])

    def wait_idx(p):
      pltpu.make_async_copy(news_r.at[pl.ds(0, CHUNK * T)],
                            idx_v[p], isem[p]).wait()

    def fire_gather(p):
      pltpu.async_copy(wtab_r.at[idx_v[p]], rows_v[p], gsem[p])

    def wait_gather(p):
      pltpu.make_async_copy(wtab_r.at[idx_v[p]], rows_v[p], gsem[p]).wait()

    def compute_out(j, p):
      rows = rows_v[p]
      acc = acc_v[p]

      def row_body(r, rcarry):
        def t_body(t, accs):
          a0, a1, a2, a3 = accs
          src = r * T + t
          a0 = a0 + rows[src, 0:16]
          a1 = a1 + rows[src, 16:32]
          a2 = a2 + rows[src, 32:48]
          a3 = a3 + rows[src, 48:64]
          return (a0, a1, a2, a3)

        z = jnp.zeros((16,), jnp.float32)
        a0, a1, a2, a3 = lax.fori_loop(0, T, t_body, (z, z, z, z), unroll=10)
        acc[r, 0:16] = a0
        acc[r, 16:32] = a1
        acc[r, 32:48] = a2
        acc[r, 48:64] = a3
        return rcarry

      lax.fori_loop(0, CHUNK, row_body, 0)
      pltpu.sync_copy(acc, wvec_r.at[pl.ds(base + j * CHUNK, CHUNK)])

    start_idx(0, 0)
    wait_idx(0)
    fire_gather(0)
    start_idx(1, 1)

    def pair_body(jj, carry):
      j0 = 2 * jj
      j1 = j0 + 1
      wait_idx(1)
      fire_gather(1)
      wait_gather(0)

      @pl.when(j0 + 2 < NCHUNK)
      def _():
        start_idx(j0 + 2, 0)

      compute_out(j0, 0)

      @pl.when(j0 + 2 < NCHUNK)
      def _():
        wait_idx(0)
        fire_gather(0)

      wait_gather(1)

      @pl.when(j1 + 2 < NCHUNK)
      def _():
        start_idx(j1 + 2, 1)

      compute_out(j1, 1)
      return carry

    lax.fori_loop(0, NCHUNK // 2, pair_body, 0)

  return body(news_flat, word_table)


def _sc_catent(dep, cat_idx, ent_idx, cat_table, ent_table):
  mesh = plsc.VectorSubcoreMesh(core_axis_name="c", subcore_axis_name="s")

  @functools.partial(
      pl.kernel,
      mesh=mesh,
      out_type=(
          jax.ShapeDtypeStruct((B, D), jnp.float32),
          jax.ShapeDtypeStruct((B, D), jnp.float32),
      ),
      compiler_params=pltpu.CompilerParams(use_tc_tiling_on_sc=False),
      scratch_types=[
          pltpu.VMEM((RPW,), jnp.int32),
          pltpu.VMEM((RPW,), jnp.int32),
          pltpu.VMEM((RPW, D), jnp.float32),
          pltpu.VMEM((RPW, D), jnp.float32),
          pltpu.SemaphoreType.DMA,
          pltpu.SemaphoreType.DMA,
      ],
  )
  def body(dep_r, cat_r, ent_r, ctab_r, etab_r, cvec_r, evec_r,
           cidx_v, eidx_v, crows_v, erows_v, isem, gsem):
    del dep_r
    wid = lax.axis_index("s") * NC + lax.axis_index("c")
    base = wid * RPW
    pltpu.async_copy(cat_r.at[pl.ds(base, RPW)], cidx_v, isem)
    pltpu.async_copy(ent_r.at[pl.ds(base, RPW)], eidx_v, isem)
    pltpu.make_async_copy(cat_r.at[pl.ds(0, RPW)], cidx_v, isem).wait()
    pltpu.make_async_copy(ent_r.at[pl.ds(0, RPW)], eidx_v, isem).wait()
    pltpu.async_copy(ctab_r.at[cidx_v], crows_v, gsem)
    pltpu.async_copy(etab_r.at[eidx_v], erows_v, gsem)
    pltpu.make_async_copy(ctab_r.at[cidx_v], crows_v, gsem).wait()
    pltpu.make_async_copy(etab_r.at[eidx_v], erows_v, gsem).wait()
    pltpu.sync_copy(crows_v, cvec_r.at[pl.ds(base, RPW)])
    pltpu.sync_copy(erows_v, evec_r.at[pl.ds(base, RPW)])

  return body(dep, cat_idx, ent_idx, cat_table, ent_table)


TC_BLK = 2048


def _tc_fuse(wsum, news, cvec, evec, W, b):
  def body(ws_r, news_r, cv_r, ev_r, w_r, b_r, out_r):
    mask = (news_r[...] != 0).astype(jnp.float32)
    cnt = jnp.sum(mask, axis=1, keepdims=True)
    wv = ws_r[...] / (cnt + 1e-08)
    dot = functools.partial(
        lax.dot_general,
        dimension_numbers=(((1,), (0,)), ((), ())),
        precision=lax.Precision.DEFAULT,
        preferred_element_type=jnp.float32,
    )
    acc = dot(wv, w_r[0:D, :])
    acc = acc + dot(cv_r[...], w_r[D:2 * D, :])
    acc = acc + dot(ev_r[...], w_r[2 * D:3 * D, :])
    out_r[...] = jnp.maximum(acc + b_r[...], 0.0)

  return pl.pallas_call(
      body,
      grid=(B // TC_BLK,),
      in_specs=[
          pl.BlockSpec((TC_BLK, D), lambda i: (i, 0)),
          pl.BlockSpec((TC_BLK, T), lambda i: (i, 0)),
          pl.BlockSpec((TC_BLK, D), lambda i: (i, 0)),
          pl.BlockSpec((TC_BLK, D), lambda i: (i, 0)),
          pl.BlockSpec((3 * D, D), lambda i: (0, 0)),
          pl.BlockSpec((1, D), lambda i: (0, 0)),
      ],
      out_specs=pl.BlockSpec((TC_BLK, D), lambda i: (i, 0)),
      out_shape=jax.ShapeDtypeStruct((B, D), jnp.float32),
  )(wsum, news, cvec, evec, W, b.reshape(1, D))


def kernel(news_input, cat_input, ent_input, word_table, cat_table, ent_table, W, b):
  news_input = news_input.astype(jnp.int32)
  news_flat = news_input.reshape(-1)
  cat_input = cat_input.astype(jnp.int32)
  ent_input = ent_input.astype(jnp.int32)
  wsum = _sc_word_pool(news_flat, word_table)
  cvec, evec = _sc_catent(wsum, cat_input, ent_input, cat_table, ent_table)
  return _tc_fuse(wsum, news_input, cvec, evec, W, b)
